# baseline (speedup 1.0000x reference)
import functools

import jax
import jax.numpy as jnp
from jax import lax
from jax.experimental import pallas as pl
from jax.experimental.pallas import tpu as pltpu
from jax.experimental.pallas import tpu_sc as plsc

_CENTER = ((1 * 3 + 1) * 3 + 1) * 2 * 3
_W_PAD = 176


def _sc_center_tap(f_pad, w_pad):
    mesh = plsc.VectorSubcoreMesh(core_axis_name="c", subcore_axis_name="s")

    @functools.partial(
        pl.kernel,
        mesh=mesh,
        out_type=jax.ShapeDtypeStruct((16,), jnp.float32),
        scratch_types=[
            pltpu.VMEM((16,), jnp.float32),
            pltpu.VMEM((_W_PAD,), jnp.float32),
            pltpu.VMEM((16,), jnp.float32),
        ],
    )
    def body(f_hbm, w_hbm, out_hbm, f_v, w_v, o_v):
        c = lax.axis_index("c")
        s = lax.axis_index("s")

        @pl.when(jnp.logical_and(c == 0, s == 0))
        def _():
            pltpu.sync_copy(f_hbm, f_v)
            pltpu.sync_copy(w_hbm, w_v)
            w0 = w_v[pl.ds(_CENTER, 16)]
            w1 = w_v[pl.ds(_CENTER + 3, 16)]
            fv = f_v[...]
            f0 = jnp.full((16,), fv[0], jnp.float32)
            f1 = jnp.full((16,), fv[1], jnp.float32)
            o_v[...] = w0 * f0 + w1 * f1
            pltpu.sync_copy(o_v, out_hbm)

    return body(f_pad, w_pad)


def kernel(features, indices, W):
    del indices
    f_pad = jnp.zeros((16,), jnp.float32).at[:2].set(features[0])
    w_pad = jnp.zeros((_W_PAD,), jnp.float32).at[:162].set(W.reshape(-1))
    out16 = _sc_center_tap(f_pad, w_pad)
    return out16[:3].reshape(1, 3)


# speedup vs baseline: 1.0666x; 1.0666x over previous
import functools

import jax
import jax.numpy as jnp
from jax import lax
from jax.experimental import pallas as pl
from jax.experimental.pallas import tpu as pltpu
from jax.experimental.pallas import tpu_sc as plsc

_WIN = 72
_OFF = 78 - _WIN


def _sc_center_tap(f_flat, w_flat):
    mesh = plsc.VectorSubcoreMesh(core_axis_name="c", subcore_axis_name="s")

    @functools.partial(
        pl.kernel,
        mesh=mesh,
        out_type=jax.ShapeDtypeStruct((1, 3), jnp.float32),
        scratch_types=[
            pltpu.VMEM((16,), jnp.float32),
            pltpu.VMEM((32,), jnp.float32),
            pltpu.VMEM((16,), jnp.float32),
        ],
    )
    def body(f_hbm, w_hbm, out_hbm, f_v, w_v, o_v):
        c = lax.axis_index("c")
        s = lax.axis_index("s")

        @pl.when(jnp.logical_and(c == 0, s == 0))
        def _():
            pltpu.sync_copy(f_hbm, f_v.at[pl.ds(0, 2)])
            pltpu.sync_copy(w_hbm.at[pl.ds(_WIN, 16)], w_v.at[pl.ds(0, 16)])
            fv = f_v[...]
            w0 = w_v[pl.ds(_OFF, 16)]
            w1 = w_v[pl.ds(_OFF + 3, 16)]
            f0 = jnp.full((16,), fv[0], jnp.float32)
            f1 = jnp.full((16,), fv[1], jnp.float32)
            o_v[...] = w0 * f0 + w1 * f1
            pltpu.sync_copy(o_v.at[pl.ds(0, 3)], out_hbm.at[0])

    return body(f_flat, w_flat)


def kernel(features, indices, W):
    del indices
    return _sc_center_tap(features.reshape(2), W.reshape(162))


# speedup vs baseline: 1.1728x; 1.0996x over previous
import functools

import jax
import jax.numpy as jnp
from jax import lax
from jax.experimental import pallas as pl
from jax.experimental.pallas import tpu as pltpu
from jax.experimental.pallas import tpu_sc as plsc

_WIN = 72
_OFF = 78 - _WIN


def _sc_center_tap(f_flat, w_flat):
    mesh = plsc.VectorSubcoreMesh(
        core_axis_name="c", subcore_axis_name="s", num_cores=1
    )

    @functools.partial(
        pl.kernel,
        mesh=mesh,
        out_type=jax.ShapeDtypeStruct((1, 3), jnp.float32),
        scratch_types=[
            pltpu.VMEM((16,), jnp.float32),
            pltpu.VMEM((32,), jnp.float32),
            pltpu.VMEM((16,), jnp.float32),
            pltpu.SemaphoreType.DMA,
            pltpu.SemaphoreType.DMA,
        ],
    )
    def body(f_hbm, w_hbm, out_hbm, f_v, w_v, o_v, sem_f, sem_w):
        c = lax.axis_index("c")
        s = lax.axis_index("s")

        @pl.when(jnp.logical_and(c == 0, s == 0))
        def _():
            cp_f = pltpu.async_copy(f_hbm, f_v.at[pl.ds(0, 2)], sem_f)
            cp_w = pltpu.async_copy(
                w_hbm.at[pl.ds(_WIN, 16)], w_v.at[pl.ds(0, 16)], sem_w
            )
            cp_f.wait()
            cp_w.wait()
            fv = f_v[...]
            w0 = w_v[pl.ds(_OFF, 16)]
            w1 = w_v[pl.ds(_OFF + 3, 16)]
            f0 = jnp.full((16,), fv[0], jnp.float32)
            f1 = jnp.full((16,), fv[1], jnp.float32)
            o_v[...] = w0 * f0 + w1 * f1
            pltpu.sync_copy(o_v.at[pl.ds(0, 3)], out_hbm.at[0])

    return body(f_flat, w_flat)


def kernel(features, indices, W):
    del indices
    return _sc_center_tap(features.reshape(2), W.reshape(162))


# speedup vs baseline: 1.1774x; 1.0039x over previous
import functools

import jax
import jax.numpy as jnp
from jax import lax
from jax.experimental import pallas as pl
from jax.experimental.pallas import tpu as pltpu
from jax.experimental.pallas import tpu_sc as plsc

_WIN = 72
_OFF = 78 - _WIN


def _sc_center_tap(f_flat, w_flat):
    mesh = plsc.VectorSubcoreMesh(
        core_axis_name="c", subcore_axis_name="s", num_cores=1, num_subcores=1
    )

    @functools.partial(
        pl.kernel,
        mesh=mesh,
        out_type=jax.ShapeDtypeStruct((1, 3), jnp.float32),
        scratch_types=[
            pltpu.VMEM((16,), jnp.float32),
            pltpu.VMEM((32,), jnp.float32),
            pltpu.VMEM((16,), jnp.float32),
            pltpu.SemaphoreType.DMA,
            pltpu.SemaphoreType.DMA,
        ],
    )
    def body(f_hbm, w_hbm, out_hbm, f_v, w_v, o_v, sem_f, sem_w):
        cp_f = pltpu.async_copy(f_hbm, f_v.at[pl.ds(0, 2)], sem_f)
        cp_w = pltpu.async_copy(
            w_hbm.at[pl.ds(_WIN, 16)], w_v.at[pl.ds(0, 16)], sem_w
        )
        cp_f.wait()
        cp_w.wait()
        fv = f_v[...]
        w0 = w_v[pl.ds(_OFF, 16)]
        w1 = w_v[pl.ds(_OFF + 3, 16)]
        f0 = jnp.full((16,), fv[0], jnp.float32)
        f1 = jnp.full((16,), fv[1], jnp.float32)
        o_v[...] = w0 * f0 + w1 * f1
        pltpu.sync_copy(o_v.at[pl.ds(0, 3)], out_hbm.at[0])

    return body(f_flat, w_flat)


def kernel(features, indices, W):
    del indices
    return _sc_center_tap(features.reshape(2), W.reshape(162))
